# Initial kernel scaffold; baseline (speedup 1.0000x reference)
import jax
import jax.numpy as jnp
from jax.experimental import pallas as pl


def kernel(input, target, Proxies, instance_label):
    raise NotImplementedError("write your pallas kernel here")


# speedup vs baseline: 33.9812x; 33.9812x over previous
import functools
import math

import jax
import jax.numpy as jnp
from jax import lax
from jax.experimental import pallas as pl
from jax.experimental.pallas import tpu as pltpu

C = 100
ALLNUM = 100000
DIM = 64
B = 1024
TOPK = 5000
SEG = 1024
NPAD = C * SEG
CT = 2048
BR = 32
INT_MIN = -2147483648
INT_MAX = 2147483647


def _key_from_boosted(boosted):
    b = lax.bitcast_convert_type(boosted, jnp.int32)
    return jnp.where(b >= 0, b, INT_MIN - b)


def _val_from_key(u):
    b = jnp.where(u >= 0, u, INT_MIN - u)
    return lax.bitcast_convert_type(b, jnp.float32)


def _k1_body(x_ref, p_ref, tgt_ref, u_ref, possum_ref):
    cb = pl.program_id(0)
    pt = p_ref[...]
    n2 = jnp.sum(pt * pt, axis=0, keepdims=True)
    invn = 1.0 / jnp.maximum(jnp.sqrt(n2), 1e-12)
    sim = jnp.dot(x_ref[...], pt,
                  preferred_element_type=jnp.float32) * invn
    j = lax.broadcasted_iota(jnp.int32, (1, CT), 1)
    cls = cb * (CT // SEG) + (j // SEG)
    ispad = (j % SEG) >= (ALLNUM // C)
    tgt = tgt_ref[...]
    pos = (cls == tgt) & jnp.logical_not(ispad)
    boosted = sim + 1000.0 * pos.astype(jnp.float32)
    u = _key_from_boosted(boosted)
    u = jnp.where(ispad, INT_MIN, u)
    u_ref[...] = u

    contrib = jnp.sum(jnp.where(pos, sim, 0.0), axis=1, keepdims=True)

    @pl.when(cb == 0)
    def _():
        possum_ref[...] = jnp.zeros_like(possum_ref)

    possum_ref[...] += contrib


def _k2_body(u_ref, tgt_ref, possum_ref, loss_ref):
    rb = pl.program_id(0)
    u = u_ref[...]

    def search_body(_, carry):
        lo, hi = carry
        half = lax.shift_right_logical(hi - lo, 1)
        mid = lo + half
        cnt = jnp.sum((u >= mid).astype(jnp.int32), axis=1, keepdims=True)
        pred = cnt >= TOPK
        return jnp.where(pred, mid, lo), jnp.where(pred, hi, mid)

    lo0 = jnp.full((BR, 1), INT_MIN, jnp.int32)
    hi0 = jnp.full((BR, 1), INT_MAX, jnp.int32)
    theta, _ = lax.fori_loop(0, 32, search_body, (lo0, hi0))

    j = lax.broadcasted_iota(jnp.int32, (1, NPAD), 1)
    cls = j // SEG
    tgt = tgt_ref[...]
    selneg = (u >= theta) & (cls != tgt)
    vals = jnp.where(selneg, _val_from_key(u), 0.0)
    logits_neg = jnp.sum(vals.reshape(BR, C, SEG), axis=2)

    c_iota = lax.broadcasted_iota(jnp.int32, (1, C), 1)
    is_t = c_iota == tgt
    logits = logits_neg + jnp.where(is_t, possum_ref[...], 0.0)

    lmask = 1.0 - (logits == 0.0).astype(jnp.float32)
    e = jnp.exp(logits) * lmask
    s = jnp.sum(e, axis=1, keepdims=True)
    e_t = jnp.sum(jnp.where(is_t, e, 0.0), axis=1, keepdims=True)
    predict_t = e_t / (1e-08 + s)
    rowloss = -jnp.log(predict_t + 1e-20)

    @pl.when(rb == 0)
    def _():
        loss_ref[...] = jnp.zeros_like(loss_ref)

    loss_ref[...] += jnp.sum(rowloss) * (1.0 / B)


@functools.partial(jax.jit, static_argnames=("interpret",))
def _run(x, target, proxies_padded, interpret=False):
    tgt2 = target.reshape(B, 1).astype(jnp.int32)
    u, possum = pl.pallas_call(
        _k1_body,
        grid=(NPAD // CT,),
        in_specs=[
            pl.BlockSpec((B, DIM), lambda cb: (0, 0)),
            pl.BlockSpec((DIM, CT), lambda cb: (0, cb)),
            pl.BlockSpec((B, 1), lambda cb: (0, 0)),
        ],
        out_specs=[
            pl.BlockSpec((B, CT), lambda cb: (0, cb)),
            pl.BlockSpec((B, 1), lambda cb: (0, 0)),
        ],
        out_shape=[
            jax.ShapeDtypeStruct((B, NPAD), jnp.int32),
            jax.ShapeDtypeStruct((B, 1), jnp.float32),
        ],
        interpret=interpret,
    )(x, proxies_padded, tgt2)

    loss = pl.pallas_call(
        _k2_body,
        grid=(B // BR,),
        in_specs=[
            pl.BlockSpec((BR, NPAD), lambda rb: (rb, 0)),
            pl.BlockSpec((BR, 1), lambda rb: (rb, 0)),
            pl.BlockSpec((BR, 1), lambda rb: (rb, 0)),
        ],
        out_specs=pl.BlockSpec((1, 1), lambda rb: (0, 0)),
        out_shape=jax.ShapeDtypeStruct((1, 1), jnp.float32),
        interpret=interpret,
    )(u, tgt2, possum)
    return loss[0, 0]


def kernel(input, target, Proxies, instance_label):
    p3 = Proxies.reshape(DIM, C, ALLNUM // C)
    p_pad = jnp.pad(p3, ((0, 0), (0, 0), (0, SEG - ALLNUM // C))).reshape(DIM, NPAD)
    loss = _run(input, target, p_pad)
    return (loss, jnp.array(0.0, dtype=jnp.float32))
